# Initial kernel scaffold; baseline (speedup 1.0000x reference)
import jax
import jax.numpy as jnp
from jax.experimental import pallas as pl


def kernel(dp, p, dl, l, Edp_emb, Eddp_emb, Edl_emb, Eddl_emb):
    raise NotImplementedError("write your pallas kernel here")


# speedup vs baseline: 1.4572x; 1.4572x over previous
import functools

import jax
import jax.numpy as jnp
from jax import lax
from jax.experimental import pallas as pl
from jax.experimental.pallas import tpu as pltpu
from jax.experimental.pallas import tpu_sc as plsc

L = 16384
D = 32
NC = 2
NS = 16
NW = NC * NS
BPW = L // NW


def _body(dp_h, p_h, dl_h, l_h, t0_h, t1_h, t2_h, t3_h, out_h,
          i0, i1, i2, i3, r0, r1, r2, r3, sem):
    wid = lax.axis_index("s") * NC + lax.axis_index("c")
    base = wid * BPW

    pltpu.sync_copy(dp_h.at[pl.ds(base, BPW)], i0)
    pltpu.sync_copy(p_h.at[pl.ds(base, BPW)], i1)
    pltpu.sync_copy(dl_h.at[pl.ds(base, BPW)], i2)
    pltpu.sync_copy(l_h.at[pl.ds(base, BPW)], i3)

    c0 = pltpu.async_copy(t0_h.at[i0], r0, sem)
    c1 = pltpu.async_copy(t1_h.at[i1], r1, sem)
    c2 = pltpu.async_copy(t2_h.at[i2], r2, sem)
    c3 = pltpu.async_copy(t3_h.at[i3], r3, sem)
    c0.wait()
    c1.wait()
    c2.wait()
    c3.wait()

    pltpu.sync_copy(r0, out_h.at[pl.ds(base, BPW), pl.ds(0 * D, D)])
    pltpu.sync_copy(r1, out_h.at[pl.ds(base, BPW), pl.ds(1 * D, D)])
    pltpu.sync_copy(r2, out_h.at[pl.ds(base, BPW), pl.ds(2 * D, D)])
    pltpu.sync_copy(r3, out_h.at[pl.ds(base, BPW), pl.ds(3 * D, D)])


@functools.partial(
    pl.kernel,
    mesh=plsc.VectorSubcoreMesh(core_axis_name="c", subcore_axis_name="s"),
    out_type=jax.ShapeDtypeStruct((L, 4 * D), jnp.float32),
    scratch_types=[
        pltpu.VMEM((BPW,), jnp.int32),
        pltpu.VMEM((BPW,), jnp.int32),
        pltpu.VMEM((BPW,), jnp.int32),
        pltpu.VMEM((BPW,), jnp.int32),
        pltpu.VMEM((BPW, D), jnp.float32),
        pltpu.VMEM((BPW, D), jnp.float32),
        pltpu.VMEM((BPW, D), jnp.float32),
        pltpu.VMEM((BPW, D), jnp.float32),
        pltpu.SemaphoreType.DMA,
    ],
    compiler_params=pltpu.CompilerParams(use_tc_tiling_on_sc=False),
)
def _hgnn_gather(dp_h, p_h, dl_h, l_h, t0_h, t1_h, t2_h, t3_h, out_h,
                 i0, i1, i2, i3, r0, r1, r2, r3, sem):
    _body(dp_h, p_h, dl_h, l_h, t0_h, t1_h, t2_h, t3_h, out_h,
          i0, i1, i2, i3, r0, r1, r2, r3, sem)


def kernel(dp, p, dl, l, Edp_emb, Eddp_emb, Edl_emb, Eddl_emb):
    dp = dp.astype(jnp.int32)
    p = p.astype(jnp.int32)
    dl = dl.astype(jnp.int32)
    l = l.astype(jnp.int32)
    out = _hgnn_gather(dp, p, dl, l, Edp_emb, Eddp_emb, Edl_emb, Eddl_emb)
    return out.reshape(1, L, 4 * D)


# speedup vs baseline: 1.5239x; 1.0458x over previous
import functools

import jax
import jax.numpy as jnp
from jax import lax
from jax.experimental import pallas as pl
from jax.experimental.pallas import tpu as pltpu
from jax.experimental.pallas import tpu_sc as plsc

L = 16384
D = 32
NC = 2
NS = 16
NW = NC * NS
BPW = L // NW


def _body(dp_h, p_h, dl_h, l_h, t0_h, t1_h, t2_h, t3_h, out_h,
          i0, i1, i2, i3, r0, r1, r2, r3,
          si0, si1, si2, si3, sg0, sg1, sg2, sg3, sw0, sw1, sw2, sw3):
    wid = lax.axis_index("s") * NC + lax.axis_index("c")
    base = wid * BPW
    idx_hs = (dp_h, p_h, dl_h, l_h)
    tbl_hs = (t0_h, t1_h, t2_h, t3_h)
    ivs = (i0, i1, i2, i3)
    rvs = (r0, r1, r2, r3)
    sis = (si0, si1, si2, si3)
    sgs = (sg0, sg1, sg2, sg3)
    sws = (sw0, sw1, sw2, sw3)

    ic = [pltpu.async_copy(idx_hs[c].at[pl.ds(base, BPW)], ivs[c], sis[c])
          for c in range(4)]
    gc = []
    for c in range(4):
        ic[c].wait()
        gc.append(pltpu.async_copy(tbl_hs[c].at[ivs[c]], rvs[c], sgs[c]))
    wc = []
    for c in range(4):
        gc[c].wait()
        wc.append(pltpu.async_copy(
            rvs[c], out_h.at[pl.ds(base, BPW), pl.ds(c * D, D)], sws[c]))
    for c in range(4):
        wc[c].wait()


@functools.partial(
    pl.kernel,
    mesh=plsc.VectorSubcoreMesh(core_axis_name="c", subcore_axis_name="s"),
    out_type=jax.ShapeDtypeStruct((L, 4 * D), jnp.float32),
    scratch_types=[
        pltpu.VMEM((BPW,), jnp.int32),
        pltpu.VMEM((BPW,), jnp.int32),
        pltpu.VMEM((BPW,), jnp.int32),
        pltpu.VMEM((BPW,), jnp.int32),
        pltpu.VMEM((BPW, D), jnp.float32),
        pltpu.VMEM((BPW, D), jnp.float32),
        pltpu.VMEM((BPW, D), jnp.float32),
        pltpu.VMEM((BPW, D), jnp.float32),
    ] + [pltpu.SemaphoreType.DMA] * 12,
    compiler_params=pltpu.CompilerParams(use_tc_tiling_on_sc=False),
)
def _hgnn_gather(dp_h, p_h, dl_h, l_h, t0_h, t1_h, t2_h, t3_h, out_h,
                 i0, i1, i2, i3, r0, r1, r2, r3, *sems):
    _body(dp_h, p_h, dl_h, l_h, t0_h, t1_h, t2_h, t3_h, out_h,
          i0, i1, i2, i3, r0, r1, r2, r3, *sems)


def kernel(dp, p, dl, l, Edp_emb, Eddp_emb, Edl_emb, Eddl_emb):
    dp = dp.astype(jnp.int32)
    p = p.astype(jnp.int32)
    dl = dl.astype(jnp.int32)
    l = l.astype(jnp.int32)
    out = _hgnn_gather(dp, p, dl, l, Edp_emb, Eddp_emb, Edl_emb, Eddl_emb)
    return out.reshape(1, L, 4 * D)


# speedup vs baseline: 1.5286x; 1.0031x over previous
import functools

import jax
import jax.numpy as jnp
from jax import lax
from jax.experimental import pallas as pl
from jax.experimental.pallas import tpu as pltpu
from jax.experimental.pallas import tpu_sc as plsc

L = 16384
D = 32
NC = 2
NS = 16
NW = NC * NS
BPW = L // NW

_MESH = plsc.VectorSubcoreMesh(core_axis_name="c", subcore_axis_name="s")
_NOTC = pltpu.CompilerParams(use_tc_tiling_on_sc=False)


def _band_body(bands, idx_hs, tbl_hs, out_h, ivs, rvs, sis, sgs, sws):
    wid = lax.axis_index("s") * NC + lax.axis_index("c")
    base = wid * BPW
    n = len(bands)
    ic = [pltpu.async_copy(idx_hs[k].at[pl.ds(base, BPW)], ivs[k], sis[k])
          for k in range(n)]
    gc = []
    for k in range(n):
        ic[k].wait()
        gc.append(pltpu.async_copy(tbl_hs[k].at[ivs[k]], rvs[k], sgs[k]))
    wc = []
    for k in range(n):
        gc[k].wait()
        wc.append(pltpu.async_copy(
            rvs[k], out_h.at[pl.ds(base, BPW), pl.ds(bands[k] * D, D)],
            sws[k]))
    for k in range(n):
        wc[k].wait()


def _make_band_kernel(bands):
    n = len(bands)
    scratch = (
        [pltpu.VMEM((BPW,), jnp.int32)] * n
        + [pltpu.VMEM((BPW, D), jnp.float32)] * n
        + [pltpu.SemaphoreType.DMA] * (3 * n)
    )

    @functools.partial(
        pl.kernel,
        mesh=_MESH,
        out_type=(),
        scratch_types=scratch,
        compiler_params=_NOTC,
        name=f"hgnn_bands_{'_'.join(map(str, bands))}",
    )
    def band_kernel(*args):
        idx_hs = args[:n]
        tbl_hs = args[n:2 * n]
        out_h = args[2 * n]
        rest = args[2 * n + 1:]
        ivs = rest[:n]
        rvs = rest[n:2 * n]
        sis = rest[2 * n:3 * n]
        sgs = rest[3 * n:4 * n]
        sws = rest[4 * n:5 * n]
        _band_body(bands, idx_hs, tbl_hs, out_h, ivs, rvs, sis, sgs, sws)

    return band_kernel


_k_small = _make_band_kernel((0, 2))
_k_big1 = _make_band_kernel((1,))
_k_big3 = _make_band_kernel((3,))


def kernel(dp, p, dl, l, Edp_emb, Eddp_emb, Edl_emb, Eddl_emb):
    dp = dp.astype(jnp.int32)
    p = p.astype(jnp.int32)
    dl = dl.astype(jnp.int32)
    l = l.astype(jnp.int32)
    out_ref = jax.new_ref(jnp.empty((L, 4 * D), jnp.float32))
    _k_small(dp, dl, Edp_emb, Edl_emb, out_ref)
    _k_big1(p, Eddp_emb, out_ref)
    _k_big3(l, Eddl_emb, out_ref)
    return out_ref[...].reshape(1, L, 4 * D)
